# Initial kernel scaffold; baseline (speedup 1.0000x reference)
import jax
import jax.numpy as jnp
from jax.experimental import pallas as pl


def kernel(x, edge_index, W, b, W1, b1, W2, b2, W3, b3):
    raise NotImplementedError("write your pallas kernel here")


# speedup vs baseline: 6.9563x; 6.9563x over previous
import functools

import jax
import jax.numpy as jnp
from jax import lax
from jax.experimental import pallas as pl
from jax.experimental.pallas import tpu as pltpu
from jax.experimental.pallas import tpu_sc as plsc

T = 4
N = 10000
E = 320000
D = 128
HID = 64
NCLS = 4

NPAD = 10240
NCORE = 2
NSUB = 16
NTILE = NCORE * NSUB
CH = 128
NCH = 80
EPT = NCH * CH
EPAD = NTILE * EPT
ZROWS = NPAD // NSUB

_mesh = plsc.VectorSubcoreMesh(core_axis_name="c", subcore_axis_name="s")


@functools.partial(
    pl.kernel,
    out_type=jax.ShapeDtypeStruct((NCORE, T, NPAD, 8), jnp.float32),
    mesh=_mesh,
    scratch_types=[
        pltpu.VMEM((NCH, CH), jnp.int32),
        pltpu.VMEM((CH, 8), jnp.float32),
        pltpu.VMEM_SHARED((NPAD, 8), jnp.float32),
    ],
)
def _deg_sc(dst_hbm, ones_hbm, z_hbm, out_hbm, dst_v, ones_v, deg_sh):
    c = lax.axis_index("c")
    s = lax.axis_index("s")
    g = c * NSUB + s
    pltpu.sync_copy(ones_hbm, ones_v)
    for t in range(T):
        pltpu.sync_copy(z_hbm, deg_sh.at[pl.ds(s * ZROWS, ZROWS)])
        plsc.subcore_barrier()
        pltpu.sync_copy(dst_hbm.at[t, g], dst_v)

        def body(i, _):
            pltpu.sync_copy(ones_v, deg_sh.at[dst_v.at[i]], add=True)
            return 0

        lax.fori_loop(0, NCH, body, 0)
        plsc.subcore_barrier()
        pltpu.sync_copy(deg_sh.at[pl.ds(s * ZROWS, ZROWS)],
                        out_hbm.at[c, t, pl.ds(s * ZROWS, ZROWS)])
        plsc.subcore_barrier()


@functools.partial(
    pl.kernel,
    out_type=jax.ShapeDtypeStruct((NCORE, T, NPAD, D), jnp.float32),
    mesh=_mesh,
    scratch_types=[
        pltpu.VMEM((NCH // 2, CH), jnp.int32),
        pltpu.VMEM((NCH // 2, CH), jnp.int32),
        pltpu.VMEM((CH, D), jnp.float32),
        pltpu.VMEM((CH, D), jnp.float32),
        pltpu.VMEM_SHARED((NPAD, D), jnp.float32),
        pltpu.SemaphoreType.DMA,
        pltpu.SemaphoreType.DMA,
    ],
)
def _agg_sc(src_hbm, dst_hbm, u_hbm, z_hbm, out_hbm,
            src_v, dst_v, rows0, rows1, acc_sh, sem0, sem1):
    c = lax.axis_index("c")
    s = lax.axis_index("s")
    g = c * NSUB + s
    hch = NCH // 2
    for t in range(T):
        pltpu.sync_copy(z_hbm, acc_sh.at[pl.ds(s * ZROWS, ZROWS)])
        plsc.subcore_barrier()
        for h in range(2):
            pltpu.sync_copy(src_hbm.at[t, g, pl.ds(h * hch, hch)], src_v)
            pltpu.sync_copy(dst_hbm.at[t, g, pl.ds(h * hch, hch)], dst_v)

            pltpu.async_copy(u_hbm.at[src_v.at[0]], rows0, sem0)

            def body(i, _):
                j0 = 2 * i
                pltpu.make_async_copy(
                    u_hbm.at[src_v.at[0]], rows0, sem0).wait()
                pltpu.async_copy(u_hbm.at[src_v.at[j0 + 1]], rows1, sem1)
                pltpu.sync_copy(rows0, acc_sh.at[dst_v.at[j0]], add=True)

                @pl.when(j0 + 2 < hch)
                def _():
                    pltpu.async_copy(u_hbm.at[src_v.at[j0 + 2]], rows0, sem0)

                pltpu.make_async_copy(
                    u_hbm.at[src_v.at[0]], rows1, sem1).wait()
                pltpu.sync_copy(rows1, acc_sh.at[dst_v.at[j0 + 1]], add=True)
                return 0

            lax.fori_loop(0, hch // 2, body, 0)
        plsc.subcore_barrier()
        pltpu.sync_copy(acc_sh.at[pl.ds(s * ZROWS, ZROWS)],
                        out_hbm.at[c, t, pl.ds(s * ZROWS, ZROWS)])
        plsc.subcore_barrier()


def _u_body(x_ref, deg_ref, u_ref):
    deg = deg_ref[0, 0, :N, 0:1] + deg_ref[1, 0, :N, 0:1] + 1.0
    u_ref[0] = x_ref[0] * (1.0 / jnp.sqrt(deg))


def _u_tc(x, deg_parts):
    return pl.pallas_call(
        _u_body,
        grid=(T,),
        in_specs=[
            pl.BlockSpec((1, N, D), lambda t: (t, 0, 0)),
            pl.BlockSpec((NCORE, 1, NPAD, 8), lambda t: (0, t, 0, 0)),
        ],
        out_specs=pl.BlockSpec((1, N, D), lambda t: (t, 0, 0)),
        out_shape=jax.ShapeDtypeStruct((T, N, D), jnp.float32),
    )(x, deg_parts)


_BN = 1000


def _head_body(agg_ref, x_ref, deg_ref, w_ref, b_ref,
               w1_ref, b1_ref, w2_ref, b2_ref, w3_ref, b3_ref, o_ref):
    deg = deg_ref[0, :, :, 0:1] + deg_ref[1, :, :, 0:1] + 1.0
    dis = 1.0 / jnp.sqrt(deg)
    acc = jnp.zeros((_BN, HID), jnp.float32)
    for t in range(T):
        v = dis[t] * (agg_ref[0, t] + agg_ref[1, t]) \
            + (dis[t] * dis[t]) * x_ref[t]
        h = jnp.dot(v, w_ref[t], preferred_element_type=jnp.float32) + b_ref[t]
        acc = acc + jnp.maximum(h, 0.0)
    h = jnp.maximum(acc, 0.0)
    h = jnp.maximum(
        jnp.dot(h, w1_ref[...], preferred_element_type=jnp.float32)
        + b1_ref[0], 0.0)
    h = jnp.maximum(
        jnp.dot(h, w2_ref[...], preferred_element_type=jnp.float32)
        + b2_ref[0], 0.0)
    o = (jnp.dot(h, w3_ref[...], preferred_element_type=jnp.float32)
         + b3_ref[0])
    mx = jnp.max(o, axis=1, keepdims=True)
    lse = mx + jnp.log(jnp.sum(jnp.exp(o - mx), axis=1, keepdims=True))
    o_ref[...] = o - lse


def _head_tc(agg_parts, x, deg_parts, W, b, W1, b1, W2, b2, W3, b3):
    return pl.pallas_call(
        _head_body,
        grid=(N // _BN,),
        in_specs=[
            pl.BlockSpec((NCORE, T, _BN, D), lambda n: (0, 0, n, 0)),
            pl.BlockSpec((T, _BN, D), lambda n: (0, n, 0)),
            pl.BlockSpec((NCORE, T, _BN, 8), lambda n: (0, 0, n, 0)),
            pl.BlockSpec((T, D, HID), lambda n: (0, 0, 0)),
            pl.BlockSpec((T, HID), lambda n: (0, 0)),
            pl.BlockSpec((HID, 32), lambda n: (0, 0)),
            pl.BlockSpec((1, 32), lambda n: (0, 0)),
            pl.BlockSpec((32, 16), lambda n: (0, 0)),
            pl.BlockSpec((1, 16), lambda n: (0, 0)),
            pl.BlockSpec((16, NCLS), lambda n: (0, 0)),
            pl.BlockSpec((1, NCLS), lambda n: (0, 0)),
        ],
        out_specs=pl.BlockSpec((_BN, NCLS), lambda n: (n, 0)),
        out_shape=jax.ShapeDtypeStruct((N, NCLS), jnp.float32),
    )(agg_parts, x, deg_parts, W, b,
      W1, b1.reshape(1, 32), W2, b2.reshape(1, 16), W3, b3.reshape(1, NCLS))


def kernel(x, edge_index, W, b, W1, b1, W2, b2, W3, b3):
    src = edge_index[:, 0, :]
    dst = edge_index[:, 1, :]
    padlen = EPAD - E
    srcp = jnp.concatenate(
        [src, jnp.zeros((T, padlen), jnp.int32)], axis=1)
    srcp = srcp + (jnp.arange(T, dtype=jnp.int32) * N)[:, None]
    dstp = jnp.concatenate(
        [dst, jnp.full((T, padlen), N, jnp.int32)], axis=1)
    src_r = srcp.reshape(T, NTILE, NCH, CH)
    dst_r = dstp.reshape(T, NTILE, NCH, CH)

    ones8 = jnp.ones((CH, 8), jnp.float32)
    z8 = jnp.zeros((ZROWS, 8), jnp.float32)
    zd = jnp.zeros((ZROWS, D), jnp.float32)

    deg0 = jax.vmap(
        lambda d_: jnp.zeros((NPAD,), jnp.float32).at[d_].add(1.0)
    )(dstp.reshape(T, EPAD))
    deg0 = jnp.broadcast_to(deg0[:, :, None], (T, NPAD, 8))
    deg_parts = jnp.stack([deg0, jnp.zeros_like(deg0)], axis=0)
    u = _u_tc(x, deg_parts)
    agg_parts = _agg_sc(src_r, dst_r, u.reshape(T * N, D), zd)
    return _head_tc(agg_parts, x, deg_parts, W, b, W1, b1, W2, b2, W3, b3)


# speedup vs baseline: 10.5787x; 1.5207x over previous
import functools

import jax
import jax.numpy as jnp
from jax import lax
from jax.experimental import pallas as pl
from jax.experimental.pallas import tpu as pltpu
from jax.experimental.pallas import tpu_sc as plsc

T = 4
N = 10000
E = 320000
D = 128
HID = 64
NCLS = 4

NPAD = 10240
NCORE = 2
NSUB = 16
NTILE = NCORE * NSUB
CH = 128
NCH = 80
EPT = NCH * CH
EPAD = NTILE * EPT
ZROWS = NPAD // NSUB

_mesh = plsc.VectorSubcoreMesh(core_axis_name="c", subcore_axis_name="s")


@functools.partial(
    pl.kernel,
    out_type=jax.ShapeDtypeStruct((NCORE, T, NPAD, D), jnp.float32),
    mesh=_mesh,
    scratch_types=[
        pltpu.VMEM((NCH // 2, CH), jnp.int32),
        pltpu.VMEM((CH, D), jnp.float32),
        pltpu.VMEM_SHARED((NPAD, D), jnp.float32),
    ],
)
def _deg_sc(dst_hbm, ones_hbm, z_hbm, out_hbm, dst_v, ones_v, deg_sh):
    c = lax.axis_index("c")
    s = lax.axis_index("s")
    g = c * NSUB + s
    hch = NCH // 2
    pltpu.sync_copy(ones_hbm, ones_v)
    for t in range(T):
        pltpu.sync_copy(z_hbm, deg_sh.at[pl.ds(s * ZROWS, ZROWS)])
        plsc.subcore_barrier()
        for h in range(2):
            pltpu.sync_copy(dst_hbm.at[t, g, pl.ds(h * hch, hch)], dst_v)

            def body(i, _):
                pltpu.sync_copy(ones_v, deg_sh.at[dst_v.at[i]], add=True)
                return 0

            lax.fori_loop(0, hch, body, 0)
        plsc.subcore_barrier()
        pltpu.sync_copy(deg_sh.at[pl.ds(s * ZROWS, ZROWS)],
                        out_hbm.at[c, t, pl.ds(s * ZROWS, ZROWS)])
        plsc.subcore_barrier()


@functools.partial(
    pl.kernel,
    out_type=jax.ShapeDtypeStruct((NCORE, T, NPAD, D), jnp.float32),
    mesh=_mesh,
    scratch_types=[
        pltpu.VMEM((NCH // 2, CH), jnp.int32),
        pltpu.VMEM((NCH // 2, CH), jnp.int32),
        pltpu.VMEM((CH, D), jnp.float32),
        pltpu.VMEM((CH, D), jnp.float32),
        pltpu.VMEM_SHARED((NPAD, D), jnp.float32),
        pltpu.SemaphoreType.DMA,
        pltpu.SemaphoreType.DMA,
    ],
)
def _agg_sc(src_hbm, dst_hbm, u_hbm, z_hbm, out_hbm,
            src_v, dst_v, rows0, rows1, acc_sh, sem0, sem1):
    c = lax.axis_index("c")
    s = lax.axis_index("s")
    g = c * NSUB + s
    hch = NCH // 2
    for t in range(T):
        pltpu.sync_copy(z_hbm, acc_sh.at[pl.ds(s * ZROWS, ZROWS)])
        plsc.subcore_barrier()
        for h in range(2):
            pltpu.sync_copy(src_hbm.at[t, g, pl.ds(h * hch, hch)], src_v)
            pltpu.sync_copy(dst_hbm.at[t, g, pl.ds(h * hch, hch)], dst_v)

            pltpu.async_copy(u_hbm.at[src_v.at[0]], rows0, sem0)

            def body(i, _):
                j0 = 2 * i
                pltpu.make_async_copy(
                    u_hbm.at[src_v.at[0]], rows0, sem0).wait()
                pltpu.async_copy(u_hbm.at[src_v.at[j0 + 1]], rows1, sem1)
                pltpu.sync_copy(rows0, acc_sh.at[dst_v.at[j0]], add=True)

                @pl.when(j0 + 2 < hch)
                def _():
                    pltpu.async_copy(u_hbm.at[src_v.at[j0 + 2]], rows0, sem0)

                pltpu.make_async_copy(
                    u_hbm.at[src_v.at[0]], rows1, sem1).wait()
                pltpu.sync_copy(rows1, acc_sh.at[dst_v.at[j0 + 1]], add=True)
                return 0

            lax.fori_loop(0, hch // 2, body, 0)
        plsc.subcore_barrier()
        pltpu.sync_copy(acc_sh.at[pl.ds(s * ZROWS, ZROWS)],
                        out_hbm.at[c, t, pl.ds(s * ZROWS, ZROWS)])
        plsc.subcore_barrier()


def _u_body(x_ref, deg_ref, u_ref):
    deg = deg_ref[0, 0, :N, 0:1] + deg_ref[1, 0, :N, 0:1] + 1.0
    u_ref[0] = x_ref[0] * (1.0 / jnp.sqrt(deg))


def _u_tc(x, deg_parts):
    return pl.pallas_call(
        _u_body,
        grid=(T,),
        in_specs=[
            pl.BlockSpec((1, N, D), lambda t: (t, 0, 0)),
            pl.BlockSpec((NCORE, 1, NPAD, D), lambda t: (0, t, 0, 0)),
        ],
        out_specs=pl.BlockSpec((1, N, D), lambda t: (t, 0, 0)),
        out_shape=jax.ShapeDtypeStruct((T, N, D), jnp.float32),
    )(x, deg_parts)


_BN = 1000


def _head_body(agg_ref, x_ref, deg_ref, w_ref, b_ref,
               w1_ref, b1_ref, w2_ref, b2_ref, w3_ref, b3_ref, o_ref):
    deg = deg_ref[0, :, :, 0:1] + deg_ref[1, :, :, 0:1] + 1.0
    dis = 1.0 / jnp.sqrt(deg)
    acc = jnp.zeros((_BN, HID), jnp.float32)
    for t in range(T):
        v = dis[t] * (agg_ref[0, t] + agg_ref[1, t]) \
            + (dis[t] * dis[t]) * x_ref[t]
        h = jnp.dot(v, w_ref[t], preferred_element_type=jnp.float32) + b_ref[t]
        acc = acc + jnp.maximum(h, 0.0)
    h = jnp.maximum(acc, 0.0)
    h = jnp.maximum(
        jnp.dot(h, w1_ref[...], preferred_element_type=jnp.float32)
        + b1_ref[0], 0.0)
    h = jnp.maximum(
        jnp.dot(h, w2_ref[...], preferred_element_type=jnp.float32)
        + b2_ref[0], 0.0)
    o = (jnp.dot(h, w3_ref[...], preferred_element_type=jnp.float32)
         + b3_ref[0])
    mx = jnp.max(o, axis=1, keepdims=True)
    lse = mx + jnp.log(jnp.sum(jnp.exp(o - mx), axis=1, keepdims=True))
    o_ref[...] = o - lse


def _head_tc(agg_parts, x, deg_parts, W, b, W1, b1, W2, b2, W3, b3):
    return pl.pallas_call(
        _head_body,
        grid=(N // _BN,),
        in_specs=[
            pl.BlockSpec((NCORE, T, _BN, D), lambda n: (0, 0, n, 0)),
            pl.BlockSpec((T, _BN, D), lambda n: (0, n, 0)),
            pl.BlockSpec((NCORE, T, _BN, D), lambda n: (0, 0, n, 0)),
            pl.BlockSpec((T, D, HID), lambda n: (0, 0, 0)),
            pl.BlockSpec((T, HID), lambda n: (0, 0)),
            pl.BlockSpec((HID, 32), lambda n: (0, 0)),
            pl.BlockSpec((1, 32), lambda n: (0, 0)),
            pl.BlockSpec((32, 16), lambda n: (0, 0)),
            pl.BlockSpec((1, 16), lambda n: (0, 0)),
            pl.BlockSpec((16, NCLS), lambda n: (0, 0)),
            pl.BlockSpec((1, NCLS), lambda n: (0, 0)),
        ],
        out_specs=pl.BlockSpec((_BN, NCLS), lambda n: (n, 0)),
        out_shape=jax.ShapeDtypeStruct((N, NCLS), jnp.float32),
    )(agg_parts, x, deg_parts, W, b,
      W1, b1.reshape(1, 32), W2, b2.reshape(1, 16), W3, b3.reshape(1, NCLS))


def kernel(x, edge_index, W, b, W1, b1, W2, b2, W3, b3):
    src = edge_index[:, 0, :]
    dst = edge_index[:, 1, :]
    padlen = EPAD - E
    srcp = jnp.concatenate(
        [src, jnp.zeros((T, padlen), jnp.int32)], axis=1)
    srcp = srcp + (jnp.arange(T, dtype=jnp.int32) * N)[:, None]
    dstp = jnp.concatenate(
        [dst, jnp.full((T, padlen), N, jnp.int32)], axis=1)
    src_r = srcp.reshape(T, NTILE, NCH, CH)
    dst_r = dstp.reshape(T, NTILE, NCH, CH)

    onesd = jnp.ones((CH, D), jnp.float32)
    zd = jnp.zeros((ZROWS, D), jnp.float32)

    deg_parts = _deg_sc(dst_r, onesd, zd)
    u = _u_tc(x, deg_parts)
    agg_parts = _agg_sc(src_r, dst_r, u.reshape(T * N, D), zd)
    return _head_tc(agg_parts, x, deg_parts, W, b, W1, b1, W2, b2, W3, b3)


# speedup vs baseline: 11.7784x; 1.1134x over previous
import functools

import jax
import jax.numpy as jnp
from jax import lax
from jax.experimental import pallas as pl
from jax.experimental.pallas import tpu as pltpu
from jax.experimental.pallas import tpu_sc as plsc

T = 4
N = 10000
E = 320000
D = 128
HID = 64
NCLS = 4

NPAD = 10240
NCORE = 2
NSUB = 16
NTILE = NCORE * NSUB
CH = 128
NCH = 80
EPT = NCH * CH
EPAD = NTILE * EPT
ZROWS = NPAD // NSUB

_mesh = plsc.VectorSubcoreMesh(core_axis_name="c", subcore_axis_name="s")


@functools.partial(
    pl.kernel,
    out_type=jax.ShapeDtypeStruct((NCORE, T, NPAD, D), jnp.float32),
    mesh=_mesh,
    scratch_types=[
        pltpu.VMEM((NCH // 2, CH), jnp.int32),
        pltpu.VMEM((CH, D), jnp.float32),
        pltpu.VMEM_SHARED((NPAD, D), jnp.float32),
    ],
)
def _deg_sc(dst_hbm, ones_hbm, z_hbm, out_hbm, dst_v, ones_v, deg_sh):
    c = lax.axis_index("c")
    s = lax.axis_index("s")
    g = c * NSUB + s
    hch = NCH // 2
    pltpu.sync_copy(ones_hbm, ones_v)
    for t in range(T):
        pltpu.sync_copy(z_hbm, deg_sh.at[pl.ds(s * ZROWS, ZROWS)])
        plsc.subcore_barrier()
        for h in range(2):
            pltpu.sync_copy(dst_hbm.at[t, g, pl.ds(h * hch, hch)], dst_v)

            def body(i, _):
                pltpu.sync_copy(ones_v, deg_sh.at[dst_v.at[i]], add=True)
                return 0

            lax.fori_loop(0, hch, body, 0)
        plsc.subcore_barrier()
        pltpu.sync_copy(deg_sh.at[pl.ds(s * ZROWS, ZROWS)],
                        out_hbm.at[c, t, pl.ds(s * ZROWS, ZROWS)])
        plsc.subcore_barrier()


@functools.partial(
    pl.kernel,
    out_type=jax.ShapeDtypeStruct((NCORE, T, NPAD, D), jnp.float32),
    mesh=_mesh,
    scratch_types=[
        pltpu.VMEM((NCH // 2, CH), jnp.int32),
        pltpu.VMEM((NCH // 2, CH), jnp.int32),
        pltpu.VMEM((CH, D), jnp.float32),
        pltpu.VMEM((CH, D), jnp.float32),
        pltpu.VMEM_SHARED((NPAD, D), jnp.float32),
        pltpu.SemaphoreType.DMA,
        pltpu.SemaphoreType.DMA,
    ],
)
def _agg_sc(src_hbm, dst_hbm, u_hbm, z_hbm, out_hbm,
            src_v, dst_v, rows0, rows1, acc_sh, sem0, sem1):
    c = lax.axis_index("c")
    s = lax.axis_index("s")
    g = c * NSUB + s
    hch = NCH // 2
    for t in range(T):
        pltpu.sync_copy(z_hbm, acc_sh.at[pl.ds(s * ZROWS, ZROWS)])
        plsc.subcore_barrier()
        for h in range(2):
            pltpu.sync_copy(src_hbm.at[t, g, pl.ds(h * hch, hch)], src_v)
            pltpu.sync_copy(dst_hbm.at[t, g, pl.ds(h * hch, hch)], dst_v)

            pltpu.async_copy(u_hbm.at[src_v.at[0]], rows0, sem0)

            def body(i, _):
                j0 = 2 * i
                pltpu.make_async_copy(
                    u_hbm.at[src_v.at[0]], rows0, sem0).wait()
                pltpu.async_copy(u_hbm.at[src_v.at[j0 + 1]], rows1, sem1)
                pltpu.sync_copy(rows0, acc_sh.at[dst_v.at[j0]], add=True)

                @pl.when(j0 + 2 < hch)
                def _():
                    pltpu.async_copy(u_hbm.at[src_v.at[j0 + 2]], rows0, sem0)

                pltpu.make_async_copy(
                    u_hbm.at[src_v.at[0]], rows1, sem1).wait()
                pltpu.sync_copy(rows1, acc_sh.at[dst_v.at[j0 + 1]], add=True)
                return 0

            lax.fori_loop(0, hch // 2, body, 0)
        plsc.subcore_barrier()
        pltpu.sync_copy(acc_sh.at[pl.ds(s * ZROWS, ZROWS)],
                        out_hbm.at[c, t, pl.ds(s * ZROWS, ZROWS)])
        plsc.subcore_barrier()


def _u_body(x_ref, deg_ref, u_ref):
    deg = deg_ref[0, 0, :N, 0:1] + deg_ref[1, 0, :N, 0:1] + 1.0
    u_ref[0] = x_ref[0] * (1.0 / jnp.sqrt(deg))


def _u_tc(x, deg_parts):
    return pl.pallas_call(
        _u_body,
        grid=(T,),
        in_specs=[
            pl.BlockSpec((1, N, D), lambda t: (t, 0, 0)),
            pl.BlockSpec((NCORE, 1, NPAD, D), lambda t: (0, t, 0, 0)),
        ],
        out_specs=pl.BlockSpec((1, N, D), lambda t: (t, 0, 0)),
        out_shape=jax.ShapeDtypeStruct((T, N, D), jnp.float32),
    )(x, deg_parts)


_BN = 1000


def _head_body(agg_ref, x_ref, deg_ref, w_ref, b_ref,
               w1_ref, b1_ref, w2_ref, b2_ref, w3_ref, b3_ref, o_ref):
    deg = deg_ref[0, :, :, 0:1] + deg_ref[1, :, :, 0:1] + 1.0
    dis = 1.0 / jnp.sqrt(deg)
    acc = jnp.zeros((_BN, HID), jnp.float32)
    for t in range(T):
        v = dis[t] * (agg_ref[0, t] + agg_ref[1, t]) \
            + (dis[t] * dis[t]) * x_ref[t]
        h = jnp.dot(v, w_ref[t], preferred_element_type=jnp.float32) + b_ref[t]
        acc = acc + jnp.maximum(h, 0.0)
    h = jnp.maximum(acc, 0.0)
    h = jnp.maximum(
        jnp.dot(h, w1_ref[...], preferred_element_type=jnp.float32)
        + b1_ref[0], 0.0)
    h = jnp.maximum(
        jnp.dot(h, w2_ref[...], preferred_element_type=jnp.float32)
        + b2_ref[0], 0.0)
    o = (jnp.dot(h, w3_ref[...], preferred_element_type=jnp.float32)
         + b3_ref[0])
    mx = jnp.max(o, axis=1, keepdims=True)
    lse = mx + jnp.log(jnp.sum(jnp.exp(o - mx), axis=1, keepdims=True))
    o_ref[...] = o - lse


def _head_tc(agg_parts, x, deg_parts, W, b, W1, b1, W2, b2, W3, b3):
    return pl.pallas_call(
        _head_body,
        grid=(N // _BN,),
        in_specs=[
            pl.BlockSpec((NCORE, T, _BN, D), lambda n: (0, 0, n, 0)),
            pl.BlockSpec((T, _BN, D), lambda n: (0, n, 0)),
            pl.BlockSpec((NCORE, T, _BN, D), lambda n: (0, 0, n, 0)),
            pl.BlockSpec((T, D, HID), lambda n: (0, 0, 0)),
            pl.BlockSpec((T, HID), lambda n: (0, 0)),
            pl.BlockSpec((HID, 32), lambda n: (0, 0)),
            pl.BlockSpec((1, 32), lambda n: (0, 0)),
            pl.BlockSpec((32, 16), lambda n: (0, 0)),
            pl.BlockSpec((1, 16), lambda n: (0, 0)),
            pl.BlockSpec((16, NCLS), lambda n: (0, 0)),
            pl.BlockSpec((1, NCLS), lambda n: (0, 0)),
        ],
        out_specs=pl.BlockSpec((_BN, NCLS), lambda n: (n, 0)),
        out_shape=jax.ShapeDtypeStruct((N, NCLS), jnp.float32),
    )(agg_parts, x, deg_parts, W, b,
      W1, b1.reshape(1, 32), W2, b2.reshape(1, 16), W3, b3.reshape(1, NCLS))


def kernel(x, edge_index, W, b, W1, b1, W2, b2, W3, b3):
    ept_real = E // NTILE
    padn = EPT - ept_real
    src = edge_index[:, 0, :].reshape(T, NTILE, ept_real)
    dst = edge_index[:, 1, :].reshape(T, NTILE, ept_real)
    pad_src = jnp.zeros((T, NTILE, padn), jnp.int32)
    pad_dst = jnp.broadcast_to(
        N + jnp.arange(padn, dtype=jnp.int32), (T, NTILE, padn))
    srcp = jnp.concatenate([src, pad_src], axis=2) \
        + (jnp.arange(T, dtype=jnp.int32) * N)[:, None, None]
    dstp = jnp.concatenate([dst, pad_dst], axis=2)
    src_r = srcp.reshape(T, NTILE, NCH, CH)
    dst_r = dstp.reshape(T, NTILE, NCH, CH)

    onesd = jnp.ones((CH, D), jnp.float32)
    zd = jnp.zeros((ZROWS, D), jnp.float32)

    deg_parts = _deg_sc(dst_r, onesd, zd)
    u = _u_tc(x, deg_parts)
    agg_parts = _agg_sc(src_r, dst_r, u.reshape(T * N, D), zd)
    return _head_tc(agg_parts, x, deg_parts, W, b, W1, b1, W2, b2, W3, b3)
